# baseline (speedup 1.0000x reference)
import functools

import jax
import jax.numpy as jnp
from jax import lax
from jax.experimental import pallas as pl
from jax.experimental.pallas import tpu as pltpu
from jax.experimental.pallas import tpu_sc as plsc

MAXLEN = 200
D = 32
B = 4096
S = 200
NW = 32
BBLK = B // NW
LANES = 16
TPITCH = BBLK + 1

_mesh = plsc.VectorSubcoreMesh(core_axis_name="c", subcore_axis_name="s")


@functools.partial(
    pl.kernel,
    mesh=_mesh,
    out_type=jax.ShapeDtypeStruct((S, D // 8, B // 128, 8, 128), jnp.float32),
    compiler_params=pltpu.CompilerParams(
        use_tc_tiling_on_sc=False, needs_layout_passes=False,
        disable_bounds_checks=True),
    scratch_types=[
        pltpu.VMEM((S, BBLK), jnp.int32),
        pltpu.VMEM((MAXLEN, D), jnp.float32),
        pltpu.VMEM((BBLK, D), jnp.float32),
        pltpu.VMEM((BBLK, D), jnp.float32),
        pltpu.VMEM((BBLK, D), jnp.float32),
        pltpu.VMEM((BBLK, D), jnp.float32),
        pltpu.VMEM((D // 8, 8, TPITCH), jnp.float32),
        pltpu.VMEM((D // 8, 8, TPITCH), jnp.float32),
        pltpu.SemaphoreType.DMA,
        pltpu.SemaphoreType.DMA,
        pltpu.SemaphoreType.DMA,
        pltpu.SemaphoreType.DMA,
        pltpu.SemaphoreType.DMA,
        pltpu.SemaphoreType.DMA,
    ],
)
def _embed(xt_hbm, tok_hbm, pos_hbm, out_hbm, idx_v, pos_v, rows0, rows1, rows2, rows3,
           tbuf0, tbuf1, sem0, sem1, sem2, sem3, st0, st1):
    wid = lax.axis_index("s") * 2 + lax.axis_index("c")
    b0 = wid * BBLK

    pltpu.sync_copy(xt_hbm.at[:, pl.ds(b0, BBLK)], idx_v)
    pltpu.sync_copy(pos_hbm, pos_v)

    rows = (rows0, rows1, rows2, rows3)
    sems = (sem0, sem1, sem2, sem3)
    tbufs = (tbuf0, tbuf1)
    sts = (st0, st1)

    lane = lax.iota(jnp.int32, 16)
    dl = lax.rem(lane, 8)
    dt0 = lax.div(lane, 8)
    dts = (dt0, dt0 + 2)
    dls = (dl, dl)

    def gather_start(s, b):
        pltpu.async_copy(tok_hbm.at[idx_v.at[s]], rows[b], sems[b])

    def gather_wait(s, b):
        pltpu.make_async_copy(tok_hbm.at[idx_v.at[s]], rows[b], sems[b]).wait()

    def store_start(s, b):
        pltpu.async_copy(tbufs[b].at[:, :, pl.ds(0, BBLK)], out_hbm.at[s, :, wid],
                         sts[b])

    def store_wait(s, b):
        pltpu.make_async_copy(tbufs[b].at[:, :, pl.ds(0, BBLK)],
                              out_hbm.at[s, :, wid], sts[b]).wait()

    for k in range(4):
        gather_start(k, k)

    def chunk_body(ss, carry):
        for b in range(4):
            s = 4 * ss + b
            tb = b % 2
            gather_wait(s, b)
            @pl.when(s >= 2)
            def _(_s=s, _tb=tb):
                store_wait(_s - 2, _tb)

            p0 = pos_v[s, pl.ds(0, LANES)]
            p1 = pos_v[s, pl.ds(LANES, LANES)]

            def row_body(i, carry2, _b=b, _tb=tb, _p0=p0, _p1=p1):
                bi = jnp.full((LANES,), i, dtype=jnp.int32)
                v0 = rows[_b][i, pl.ds(0, LANES)] + _p0
                plsc.store_scatter(tbufs[_tb], [dts[0], dls[0], bi], v0)
                v1 = rows[_b][i, pl.ds(LANES, LANES)] + _p1
                plsc.store_scatter(tbufs[_tb], [dts[1], dls[1], bi], v1)
                return carry2

            lax.fori_loop(0, BBLK, row_body, 0, unroll=4)
            store_start(s, tb)

            @pl.when(s + 4 < S)
            def _(_s=s, _b=b):
                gather_start(_s + 4, _b)

        return carry

    lax.fori_loop(0, S // 4, chunk_body, 0)
    store_wait(S - 2, 0)
    store_wait(S - 1, 1)


def kernel(x, token_table, pos_table):
    xt = x.astype(jnp.int32).T
    out5 = _embed(xt, token_table, pos_table)
    return out5.transpose(2, 4, 0, 1, 3).reshape(B, S, D)


# speedup vs baseline: 1.0052x; 1.0052x over previous
import functools

import jax
import jax.numpy as jnp
from jax import lax
from jax.experimental import pallas as pl
from jax.experimental.pallas import tpu as pltpu
from jax.experimental.pallas import tpu_sc as plsc

MAXLEN = 200
D = 32
B = 4096
S = 200
NW = 32
BBLK = B // NW
LANES = 16
TPITCH = BBLK + 1

_mesh = plsc.VectorSubcoreMesh(core_axis_name="c", subcore_axis_name="s")


@functools.partial(
    pl.kernel,
    mesh=_mesh,
    out_type=jax.ShapeDtypeStruct((S, D // 8, B // 128, 8, 128), jnp.float32),
    compiler_params=pltpu.CompilerParams(
        use_tc_tiling_on_sc=False, needs_layout_passes=False,
        disable_bounds_checks=True),
    scratch_types=[
        pltpu.VMEM((S, BBLK), jnp.int32),
        pltpu.VMEM((MAXLEN, D), jnp.float32),
        pltpu.VMEM((BBLK, D), jnp.float32),
        pltpu.VMEM((BBLK, D), jnp.float32),
        pltpu.VMEM((BBLK, D), jnp.float32),
        pltpu.VMEM((BBLK, D), jnp.float32),
        pltpu.VMEM((D // 8, 8, TPITCH), jnp.float32),
        pltpu.VMEM((D // 8, 8, TPITCH), jnp.float32),
        pltpu.SemaphoreType.DMA,
        pltpu.SemaphoreType.DMA,
        pltpu.SemaphoreType.DMA,
        pltpu.SemaphoreType.DMA,
        pltpu.SemaphoreType.DMA,
        pltpu.SemaphoreType.DMA,
    ],
)
def _embed(xt_hbm, tok_hbm, pos_hbm, out_hbm, idx_v, pos_v, rows0, rows1, rows2, rows3,
           tbuf0, tbuf1, sem0, sem1, sem2, sem3, st0, st1):
    wid = lax.axis_index("s") * 2 + lax.axis_index("c")
    b0 = wid * BBLK

    pltpu.sync_copy(xt_hbm.at[:, pl.ds(b0, BBLK)], idx_v)
    pltpu.sync_copy(pos_hbm, pos_v)

    rows = (rows0, rows1, rows2, rows3)
    sems = (sem0, sem1, sem2, sem3)
    tbufs = (tbuf0, tbuf1)
    sts = (st0, st1)

    lane = lax.iota(jnp.int32, 16)
    dl = lax.rem(lane, 8)
    dt0 = lax.div(lane, 8)
    dts = (dt0, dt0 + 2)
    dls = (dl, dl)

    def gather_start(s, b):
        pltpu.async_copy(tok_hbm.at[idx_v.at[s]], rows[b], sems[b])

    def gather_wait(s, b):
        pltpu.make_async_copy(tok_hbm.at[idx_v.at[s]], rows[b], sems[b]).wait()

    def store_start(s, b):
        pltpu.async_copy(tbufs[b].at[:, :, pl.ds(0, BBLK)], out_hbm.at[s, :, wid],
                         sts[b])

    def store_wait(s, b):
        pltpu.make_async_copy(tbufs[b].at[:, :, pl.ds(0, BBLK)],
                              out_hbm.at[s, :, wid], sts[b]).wait()

    for k in range(4):
        gather_start(k, k)

    def chunk_body(ss, carry):
        for b in range(4):
            s = 4 * ss + b
            tb = b % 2
            gather_wait(s, b)
            @pl.when(s >= 2)
            def _(_s=s, _tb=tb):
                store_wait(_s - 2, _tb)

            p0 = pos_v[s, pl.ds(0, LANES)]
            p1 = pos_v[s, pl.ds(LANES, LANES)]

            def row_body(i, carry2, _b=b, _tb=tb, _p0=p0, _p1=p1):
                bi = jnp.full((LANES,), i, dtype=jnp.int32)
                v0 = rows[_b][i, pl.ds(0, LANES)] + _p0
                plsc.store_scatter(tbufs[_tb], [dts[0], dls[0], bi], v0)
                v1 = rows[_b][i, pl.ds(LANES, LANES)] + _p1
                plsc.store_scatter(tbufs[_tb], [dts[1], dls[1], bi], v1)
                return carry2

            lax.fori_loop(0, BBLK, row_body, 0, unroll=8)
            store_start(s, tb)

            @pl.when(s + 4 < S)
            def _(_s=s, _b=b):
                gather_start(_s + 4, _b)

        return carry

    lax.fori_loop(0, S // 4, chunk_body, 0)
    store_wait(S - 2, 0)
    store_wait(S - 1, 1)


def kernel(x, token_table, pos_table):
    xt = x.astype(jnp.int32).T
    out5 = _embed(xt, token_table, pos_table)
    return out5.transpose(2, 4, 0, 1, 3).reshape(B, S, D)
